# baseline (speedup 1.0000x reference)
import jax
import jax.numpy as jnp
from jax import lax
from jax.experimental import pallas as pl
from jax.experimental.pallas import tpu as pltpu
from jax.experimental.pallas import tpu_sc as plsc

EMBED = 64
SEQ = 200
BATCH = 4096
NW = 32
BPW = BATCH // NW
NB = 3
LANES = 16


def _body(xt_hbm, tok_hbm, pos_hbm, out_hbm, idx_v, pos_v, rows_v, t_v,
          gsem0, gsem1, gsem2, osem0, osem1, osem2):
    gsems = (gsem0, gsem1, gsem2)
    osems = (osem0, osem1, osem2)
    wid = lax.axis_index("s") * 2 + lax.axis_index("c")

    pltpu.sync_copy(xt_hbm.at[:, pl.ds(wid * BPW, BPW)], idx_v)
    pltpu.sync_copy(pos_hbm, pos_v)

    iota = lax.iota(jnp.int32, LANES)

    def fire_gather(s, b):
        pltpu.async_copy(tok_hbm.at[idx_v.at[s]],
                         rows_v.at[pl.ds(b * BPW, BPW)], gsems[b])

    def drain_gather(s, b):
        pltpu.make_async_copy(tok_hbm.at[idx_v.at[s]],
                              rows_v.at[pl.ds(b * BPW, BPW)],
                              gsems[b]).wait()

    def fire_out(s, b):
        for fh in range(EMBED // 8):
            pltpu.async_copy(t_v.at[b, fh], out_hbm.at[s, fh, wid],
                             osems[b])

    def drain_out(s, b):
        for fh in range(EMBED // 8):
            pltpu.make_async_copy(t_v.at[b, fh], out_hbm.at[s, fh, wid],
                                  osems[b]).wait()

    def transpose_add(s, b):
        base = b * BPW
        for c in range(EMBED // LANES):
            p16 = pos_v[s, pl.ds(c * LANES, LANES)]
            for lane in range(LANES):
                fo = c * LANES + lane
                splat = jnp.take_along_axis(
                    p16, jnp.full((LANES,), lane, jnp.int32), axis=0)
                cidx = jnp.full((LANES,), fo, jnp.int32)
                for jj in range(BPW // LANES):
                    ridx = iota + (base + jj * LANES)
                    vals = plsc.load_gather(rows_v, [ridx, cidx])
                    t_v[b, fo // 8, fo % 8, pl.ds(jj * LANES, LANES)] = (
                        vals + splat)

    for b in range(NB):
        fire_gather(b, b)

    def outer(i, carry):
        for b in range(NB):
            s = i * NB + b

            @pl.when(s < SEQ)
            def _process():
                drain_gather(s, b)

                @pl.when(s >= NB)
                def _reclaim():
                    drain_out(s - NB, b)

                transpose_add(s, b)
                fire_out(s, b)

                @pl.when(s + NB < SEQ)
                def _ahead():
                    fire_gather(s + NB, b)
        return carry

    lax.fori_loop(0, (SEQ + NB - 1) // NB, outer, 0)
    for b in range(NB):
        s_last = max(s for s in range(SEQ) if s % NB == b)
        drain_out(s_last, b)


def _impl(xt, tok, pos):
    mesh = plsc.VectorSubcoreMesh(core_axis_name="c", subcore_axis_name="s")
    f = pl.kernel(
        _body,
        mesh=mesh,
        out_type=jax.ShapeDtypeStruct(
            (SEQ, EMBED // 8, BATCH // 128, 8, 128), jnp.float32),
        scratch_types=[
            pltpu.VMEM((SEQ, BPW), jnp.int32),
            pltpu.VMEM((SEQ, EMBED), jnp.float32),
            pltpu.VMEM((NB * BPW, EMBED), jnp.float32),
            pltpu.VMEM((NB, EMBED // 8, 8, 128), jnp.float32),
            pltpu.SemaphoreType.DMA,
            pltpu.SemaphoreType.DMA,
            pltpu.SemaphoreType.DMA,
            pltpu.SemaphoreType.DMA,
            pltpu.SemaphoreType.DMA,
            pltpu.SemaphoreType.DMA,
        ],
        compiler_params=pltpu.CompilerParams(use_tc_tiling_on_sc=False,
                                             needs_layout_passes=False),
    )
    return f(xt, tok, pos)


def kernel(x, token_table, pos_table):
    batch, seq = x.shape
    xt = jnp.swapaxes(x.astype(jnp.int32), 0, 1)
    out5 = _impl(xt, token_table, pos_table)
    return out5.transpose(2, 4, 0, 1, 3).reshape(batch, seq, EMBED)


# speedup vs baseline: 1.7884x; 1.7884x over previous
import jax
import jax.numpy as jnp
from jax import lax
from jax.experimental import pallas as pl
from jax.experimental.pallas import tpu as pltpu
from jax.experimental.pallas import tpu_sc as plsc

EMBED = 64
SEQ = 200
BATCH = 4096
NW = 32
BPW = BATCH // NW
NB = 3
LANES = 16


def _body(xt_hbm, tok_hbm, pos_hbm, out_hbm, idx_v, pos_v, rows_v, t_v,
          gsem0, gsem1, gsem2, osem0, osem1, osem2):
    gsems = (gsem0, gsem1, gsem2)
    osems = (osem0, osem1, osem2)
    wid = lax.axis_index("s") * 2 + lax.axis_index("c")

    pltpu.sync_copy(xt_hbm.at[:, pl.ds(wid * BPW, BPW)], idx_v)
    pltpu.sync_copy(pos_hbm, pos_v)

    iota = lax.iota(jnp.int32, LANES)

    def fire_gather(s, b):
        pltpu.async_copy(tok_hbm.at[idx_v.at[s]],
                         rows_v.at[pl.ds(b * BPW, BPW)], gsems[b])

    def drain_gather(s, b):
        pltpu.make_async_copy(tok_hbm.at[idx_v.at[s]],
                              rows_v.at[pl.ds(b * BPW, BPW)],
                              gsems[b]).wait()

    def fire_out(s, b):
        for fh in range(EMBED // 8):
            pltpu.async_copy(t_v.at[b, pl.ds(fh * 8, 8), pl.ds(0, BPW)],
                             out_hbm.at[s, fh, wid], osems[b])

    def drain_out(s, b):
        for fh in range(EMBED // 8):
            pltpu.make_async_copy(t_v.at[b, pl.ds(fh * 8, 8), pl.ds(0, BPW)],
                                  out_hbm.at[s, fh, wid],
                                  osems[b]).wait()

    def transpose_add(s, b):
        base = b * BPW
        bvec = jnp.full((LANES,), b, jnp.int32)
        ps = [pos_v[s, pl.ds(c * LANES, LANES)]
              for c in range(EMBED // LANES)]
        ridx = [iota + c * LANES for c in range(EMBED // LANES)]
        unroll = 16

        def tok_block(k, carry):
            for u in range(unroll):
                bb = k * unroll + u
                cvec = jnp.full((LANES,), bb, jnp.int32)
                for c in range(EMBED // LANES):
                    v = rows_v[base + bb, pl.ds(c * LANES, LANES)] + ps[c]
                    plsc.store_scatter(t_v, [bvec, ridx[c], cvec], v)
            return carry

        lax.fori_loop(0, BPW // unroll, tok_block, 0)

    for b in range(NB):
        fire_gather(b, b)

    def outer(i, carry):
        for b in range(NB):
            s = i * NB + b

            @pl.when(s < SEQ)
            def _process():
                drain_gather(s, b)

                @pl.when(s >= NB)
                def _reclaim():
                    drain_out(s - NB, b)

                transpose_add(s, b)
                fire_out(s, b)

                @pl.when(s + NB < SEQ)
                def _ahead():
                    fire_gather(s + NB, b)
        return carry

    lax.fori_loop(0, (SEQ + NB - 1) // NB, outer, 0)
    for b in range(NB):
        s_last = max(s for s in range(SEQ) if s % NB == b)
        drain_out(s_last, b)


def _impl(xt, tok, pos):
    mesh = plsc.VectorSubcoreMesh(core_axis_name="c", subcore_axis_name="s")
    f = pl.kernel(
        _body,
        mesh=mesh,
        out_type=jax.ShapeDtypeStruct(
            (SEQ, EMBED // 8, BATCH // 128, 8, 128), jnp.float32),
        scratch_types=[
            pltpu.VMEM((SEQ, BPW), jnp.int32),
            pltpu.VMEM((SEQ, EMBED), jnp.float32),
            pltpu.VMEM((NB * BPW, EMBED), jnp.float32),
            pltpu.VMEM((NB, EMBED, 129), jnp.float32),
            pltpu.SemaphoreType.DMA,
            pltpu.SemaphoreType.DMA,
            pltpu.SemaphoreType.DMA,
            pltpu.SemaphoreType.DMA,
            pltpu.SemaphoreType.DMA,
            pltpu.SemaphoreType.DMA,
        ],
        compiler_params=pltpu.CompilerParams(use_tc_tiling_on_sc=False,
                                             needs_layout_passes=False),
    )
    return f(xt, tok, pos)


def kernel(x, token_table, pos_table):
    batch, seq = x.shape
    xt = jnp.swapaxes(x.astype(jnp.int32), 0, 1)
    out5 = _impl(xt, token_table, pos_table)
    return out5.transpose(2, 4, 0, 1, 3).reshape(batch, seq, EMBED)
